# baseline (speedup 1.0000x reference)
import functools

import jax
import jax.numpy as jnp
from jax import lax
from jax.experimental import pallas as pl
from jax.experimental.pallas import tpu as pltpu
from jax.experimental.pallas import tpu_sc as plsc

ROWS = 128
COLS = 32768
NUM_CORES = 2
NUM_SUBCORES = 16
LANES = 16
NUM_WORKERS = NUM_CORES * NUM_SUBCORES
ROWS_PER_WORKER = ROWS // NUM_WORKERS
VREGS = COLS // LANES
UNROLL = 8
NBUF = 3


def _bcast_last(v, last_idx):
  return jnp.take(v, last_idx)


def _scan_row(buf, last_idx):
  zero = jnp.zeros((LANES,), jnp.float32)

  def body(g, carry):
    vs = [buf[pl.ds((g + u) * LANES, LANES)] for u in range(UNROLL)]
    scans = [plsc.cumsum(v) for v in vs]
    totals = [_bcast_last(s, last_idx) for s in scans]
    pt = [totals[0]]
    for u in range(1, UNROLL):
      pt.append(pt[u - 1] + totals[u])
    outs = [carry + scans[0]]
    for u in range(1, UNROLL):
      outs.append((carry + pt[u - 1]) + scans[u])
    for u in range(UNROLL):
      buf[pl.ds((g + u) * LANES, LANES)] = outs[u]
    return carry + pt[UNROLL - 1]

  plsc.parallel_loop(0, VREGS, step=UNROLL, carry=zero)(body)


def _body(x_hbm, out_hbm, b0, b1, b2, si0, si1, si2, so0, so1, so2):
  bufs = (b0, b1, b2)
  sin = (si0, si1, si2)
  sout = (so0, so1, so2)
  wid = lax.axis_index("s") * NUM_CORES + lax.axis_index("c")
  base = wid * ROWS_PER_WORKER
  last_idx = jnp.full((LANES,), LANES - 1, jnp.int32)

  ins = [
      pltpu.async_copy(x_hbm.at[base + c], bufs[c], sin[c])
      for c in range(min(NBUF, ROWS_PER_WORKER))
  ]
  outs = [None] * ROWS_PER_WORKER
  out_waited = [False] * ROWS_PER_WORKER
  for c in range(ROWS_PER_WORKER):
    s = c % NBUF
    ins[c].wait()
    _scan_row(bufs[s], last_idx)
    outs[c] = pltpu.async_copy(bufs[s], out_hbm.at[base + c], sout[s])
    nxt = c + 2
    if c >= 1 and nxt < ROWS_PER_WORKER:
      outs[c - 1].wait()
      out_waited[c - 1] = True
      ins.append(
          pltpu.async_copy(x_hbm.at[base + nxt], bufs[nxt % NBUF],
                           sin[nxt % NBUF]))
  for c in range(ROWS_PER_WORKER):
    if not out_waited[c]:
      outs[c].wait()


_cumsum_sc = functools.partial(
    pl.kernel,
    out_type=jax.ShapeDtypeStruct((ROWS, COLS), jnp.float32),
    mesh=plsc.VectorSubcoreMesh(core_axis_name="c", subcore_axis_name="s"),
    scratch_types=[
        pltpu.VMEM((COLS,), jnp.float32),
        pltpu.VMEM((COLS,), jnp.float32),
        pltpu.VMEM((COLS,), jnp.float32),
        pltpu.SemaphoreType.DMA,
        pltpu.SemaphoreType.DMA,
        pltpu.SemaphoreType.DMA,
        pltpu.SemaphoreType.DMA,
        pltpu.SemaphoreType.DMA,
        pltpu.SemaphoreType.DMA,
    ],
    compiler_params=pltpu.CompilerParams(needs_layout_passes=False),
)(_body)


def kernel(x):
  return _cumsum_sc(x)
